# baseline (speedup 1.0000x reference)
import functools

import jax
import jax.numpy as jnp
from jax import lax
from jax.experimental import pallas as pl
from jax.experimental.pallas import tpu as pltpu
from jax.experimental.pallas import tpu_sc as plsc

H = 768
HC = H // 16
EPS = 1e-12
NC, NS = 2, 16
NW = NC * NS
PC = 32


def _build_ctab(word_pad, type_emb):
    def body(w_ref, t_ref, o_ref):
        o_ref[...] = w_ref[...][None] + t_ref[...][:, None, :]

    return pl.pallas_call(
        body,
        out_shape=jax.ShapeDtypeStruct((2, 32, H), jnp.float32),
    )(word_pad, type_emb)


def _rsqrt16(x):
    i = plsc.bitcast(x, jnp.int32)
    i = jnp.int32(0x5F3759DF) - lax.shift_right_logical(i, 1)
    y = plsc.bitcast(i, jnp.float32)
    for _ in range(3):
        y = y * (1.5 - 0.5 * x * y * y)
    return y


def _make_sc_kernel(B, L):
    PPW = L // NW
    NCH = PPW // PC
    mesh = plsc.VectorSubcoreMesh(
        core_axis_name="c", subcore_axis_name="s", num_cores=NC, num_subcores=NS
    )

    @functools.partial(
        pl.kernel,
        out_type=jax.ShapeDtypeStruct((B * L * H,), jnp.float32),
        mesh=mesh,
        scratch_types=[
            pltpu.VMEM((64 * H,), jnp.float32),
            pltpu.VMEM((PC * H,), jnp.float32),
            pltpu.VMEM((PC * H,), jnp.float32),
            pltpu.VMEM((B * PPW,), jnp.int32),
            pltpu.VMEM((H,), jnp.float32),
            pltpu.VMEM((H,), jnp.float32),
        ],
        compiler_params=pltpu.CompilerParams(needs_layout_passes=False),
    )
    def emb_ln(cid_hbm, ctab_hbm, pos_hbm, gam_hbm, bet_hbm, out_hbm,
               ctab_v, pos_v, obuf_v, cid_v, gam_v, bet_v):
        wid = lax.axis_index("s") * NC + lax.axis_index("c")
        p_base = wid * PPW
        pltpu.sync_copy(ctab_hbm, ctab_v)
        pltpu.sync_copy(gam_hbm, gam_v)
        pltpu.sync_copy(bet_hbm, bet_v)
        for b in range(B):
            pltpu.sync_copy(
                cid_hbm.at[pl.ds(b * L + p_base, PPW)],
                cid_v.at[pl.ds(b * PPW, PPW)],
            )
        iota = lax.iota(jnp.int32, 16)

        def chunk_body(ci, carry):
            p0 = p_base + ci * PC
            pltpu.sync_copy(pos_hbm.at[pl.ds(p0 * H, PC * H)], pos_v)

            def batch_body(b, carry):
                def tok_body(t, carry):
                    tloc = b * PPW + ci * PC + t
                    cid16 = plsc.load_gather(
                        cid_v, [jnp.full((16,), tloc, jnp.int32)]
                    )
                    idx = cid16 * H + iota
                    obase = t * H
                    acc = [jnp.zeros((16,), jnp.float32) for _ in range(4)]
                    accq = [jnp.zeros((16,), jnp.float32) for _ in range(4)]
                    for k in range(HC):
                        w = plsc.load_gather(ctab_v, [idx])
                        p = pos_v[pl.ds(obase + k * 16, 16)]
                        x = w + p
                        obuf_v[pl.ds(obase + k * 16, 16)] = x
                        acc[k % 4] = acc[k % 4] + x
                        accq[k % 4] = accq[k % 4] + x * x
                        idx = idx + 16
                    s = (acc[0] + acc[1]) + (acc[2] + acc[3])
                    q = (accq[0] + accq[1]) + (accq[2] + accq[3])
                    mean = jnp.broadcast_to(jnp.sum(s), (16,)) * (1.0 / H)
                    msq = jnp.broadcast_to(jnp.sum(q), (16,)) * (1.0 / H)
                    istd = _rsqrt16(msq - mean * mean + EPS)
                    for k in range(HC):
                        x = obuf_v[pl.ds(obase + k * 16, 16)]
                        y = (x - mean) * istd
                        o = y * gam_v[pl.ds(k * 16, 16)] + bet_v[pl.ds(k * 16, 16)]
                        obuf_v[pl.ds(obase + k * 16, 16)] = o
                    return carry

                lax.fori_loop(0, PC, tok_body, carry)
                pltpu.sync_copy(
                    obuf_v, out_hbm.at[pl.ds((b * L + p0) * H, PC * H)]
                )
                return carry

            return lax.fori_loop(0, B, batch_body, carry)

        lax.fori_loop(0, NCH, chunk_body, 0)

    return emb_ln


def kernel(input_ids, token_type_ids, word_emb, pos_emb, type_emb, ln_gamma, ln_beta):
    B, L = input_ids.shape
    cid = (input_ids + 32 * token_type_ids).reshape(-1)
    word_pad = jnp.pad(word_emb, ((0, 32 - word_emb.shape[0]), (0, 0)))
    ctab = _build_ctab(word_pad, type_emb).reshape(-1)
    out = _make_sc_kernel(B, L)(
        cid, ctab, pos_emb.reshape(-1), ln_gamma, ln_beta
    )
    return out.reshape(B, L, H)
